# baseline (speedup 1.0000x reference)
import functools

import jax
import jax.numpy as jnp
from jax.experimental import pallas as pl
from jax.experimental.pallas import tpu as pltpu

E = 8
TOP_K = 2
RANK = 8
ALPHA = 16.0
D_IN = 2048
D_OUT = 2048
ER = E * RANK

_BLK = 512


def _body(x_ref, w_ref, b_ref, rw_ref, a_ref, bm_ref, o_ref, w16_ref):
    @pl.when(pl.program_id(0) == 0)
    def _fill():
        w16_ref[...] = w_ref[...].astype(jnp.bfloat16)

    xb16 = x_ref[...].astype(jnp.bfloat16)
    dn_t = (((1,), (1,)), ((), ()))
    dn_l = (((0,), (0,)), ((), ()))

    lt = jax.lax.dot_general(rw_ref[...].astype(jnp.bfloat16), xb16, dn_t,
                             preferred_element_type=jnp.float32)
    riota = jax.lax.broadcasted_iota(jnp.int32, (E, _BLK), 0)
    v1 = jnp.max(lt, axis=0, keepdims=True)
    i1 = jnp.min(jnp.where(lt == v1, riota, E), axis=0, keepdims=True)
    masked = jnp.where(riota == i1, -jnp.inf, lt)
    v2 = jnp.max(masked, axis=0, keepdims=True)
    i2 = jnp.min(jnp.where(masked == v2, riota, E), axis=0, keepdims=True)
    ee = jnp.exp(v2 - v1)
    denom = 1.0 + ee
    p1 = (ALPHA / RANK) / denom
    p2 = (ALPHA / RANK) * ee / denom

    h = jax.lax.dot_general(a_ref[...].astype(jnp.bfloat16), xb16, dn_t,
                            preferred_element_type=jnp.float32)
    row = jax.lax.broadcasted_iota(jnp.int32, (ER, _BLK), 0) // RANK
    g = jnp.where(row == i1, p1, 0.0) + jnp.where(row == i2, p2, 0.0)
    hw = (h * g).astype(jnp.bfloat16)

    fused = jax.lax.dot_general(hw, bm_ref[...].astype(jnp.bfloat16), dn_l,
                                preferred_element_type=jnp.float32)

    w16 = w16_ref[...]
    bias = b_ref[...]
    half = _BLK // 2
    y0 = jax.lax.dot_general(xb16[:half], w16, dn_t,
                             preferred_element_type=jnp.float32)
    o_ref[:half] = y0 + (fused[:half] + bias)
    y1 = jax.lax.dot_general(xb16[half:], w16, dn_t,
                             preferred_element_type=jnp.float32)
    o_ref[half:] = y1 + (fused[half:] + bias)


@jax.jit
def kernel(x, base_W, base_b, router_W, A, Bm):
    b, s, _ = x.shape
    n = b * s
    x2 = x.reshape(n, D_IN)
    a_flat = A.reshape(ER, D_IN)
    bm_flat = jnp.transpose(Bm, (0, 2, 1)).reshape(ER, D_OUT)
    bias = base_b.reshape(1, D_OUT)

    grid = (n // _BLK,)
    out = pl.pallas_call(
        _body,
        grid=grid,
        in_specs=[
            pl.BlockSpec((_BLK, D_IN), lambda i: (i, 0)),
            pl.BlockSpec((D_OUT, D_IN), lambda i: (0, 0)),
            pl.BlockSpec((1, D_OUT), lambda i: (0, 0)),
            pl.BlockSpec((E, D_IN), lambda i: (0, 0)),
            pl.BlockSpec((ER, D_IN), lambda i: (0, 0)),
            pl.BlockSpec((ER, D_OUT), lambda i: (0, 0)),
        ],
        out_specs=pl.BlockSpec((_BLK, D_OUT), lambda i: (i, 0)),
        out_shape=jax.ShapeDtypeStruct((n, D_OUT), jnp.float32),
        scratch_shapes=[pltpu.VMEM((D_OUT, D_IN), jnp.bfloat16)],
        compiler_params=pltpu.CompilerParams(vmem_limit_bytes=64 * 1024 * 1024),
    )(x2, base_W, bias, router_W, a_flat, bm_flat)
    return out.reshape(b, s, D_OUT)


# speedup vs baseline: 1.0111x; 1.0111x over previous
import functools

import jax
import jax.numpy as jnp
from jax.experimental import pallas as pl
from jax.experimental.pallas import tpu as pltpu

E = 8
TOP_K = 2
RANK = 8
ALPHA = 16.0
D_IN = 2048
D_OUT = 2048
ER = E * RANK

_BLK = 1024


def _body(x_ref, w_ref, b_ref, rw_ref, a_ref, bm_ref, o_ref):
    xb16 = x_ref[...].astype(jnp.bfloat16)
    dn_t = (((1,), (1,)), ((), ()))
    dn_l = (((0,), (0,)), ((), ()))

    lt = jax.lax.dot_general(rw_ref[...].astype(jnp.bfloat16), xb16, dn_t,
                             preferred_element_type=jnp.float32)
    riota = jax.lax.broadcasted_iota(jnp.int32, (E, _BLK), 0)
    v1 = jnp.max(lt, axis=0, keepdims=True)
    i1 = jnp.min(jnp.where(lt == v1, riota, E), axis=0, keepdims=True)
    masked = jnp.where(riota == i1, -jnp.inf, lt)
    v2 = jnp.max(masked, axis=0, keepdims=True)
    i2 = jnp.min(jnp.where(masked == v2, riota, E), axis=0, keepdims=True)
    ee = jnp.exp(v2 - v1)
    denom = 1.0 + ee
    p1 = (ALPHA / RANK) / denom
    p2 = (ALPHA / RANK) * ee / denom

    h = jax.lax.dot_general(a_ref[...].astype(jnp.bfloat16), xb16, dn_t,
                            preferred_element_type=jnp.float32)
    row = jax.lax.broadcasted_iota(jnp.int32, (ER, _BLK), 0) // RANK
    g = jnp.where(row == i1, p1, 0.0) + jnp.where(row == i2, p2, 0.0)
    hw = (h * g).astype(jnp.bfloat16)

    w16 = w_ref[...].astype(jnp.bfloat16)
    bm16 = bm_ref[...].astype(jnp.bfloat16)
    bias = b_ref[...]
    half = _BLK // 2
    f0 = jax.lax.dot_general(hw[:, :half], bm16, dn_l,
                             preferred_element_type=jnp.float32)
    y0 = jax.lax.dot_general(xb16[:half], w16, dn_t,
                             preferred_element_type=jnp.float32)
    o_ref[:half] = y0 + (f0 + bias)
    f1 = jax.lax.dot_general(hw[:, half:], bm16, dn_l,
                             preferred_element_type=jnp.float32)
    y1 = jax.lax.dot_general(xb16[half:], w16, dn_t,
                             preferred_element_type=jnp.float32)
    o_ref[half:] = y1 + (f1 + bias)


@jax.jit
def kernel(x, base_W, base_b, router_W, A, Bm):
    b, s, _ = x.shape
    n = b * s
    x2 = x.reshape(n, D_IN)
    a_flat = A.reshape(ER, D_IN)
    bm_flat = jnp.transpose(Bm, (0, 2, 1)).reshape(ER, D_OUT)
    bias = base_b.reshape(1, D_OUT)

    grid = (n // _BLK,)
    out = pl.pallas_call(
        _body,
        grid=grid,
        in_specs=[
            pl.BlockSpec((_BLK, D_IN), lambda i: (i, 0)),
            pl.BlockSpec((D_OUT, D_IN), lambda i: (0, 0)),
            pl.BlockSpec((1, D_OUT), lambda i: (0, 0)),
            pl.BlockSpec((E, D_IN), lambda i: (0, 0)),
            pl.BlockSpec((ER, D_IN), lambda i: (0, 0)),
            pl.BlockSpec((ER, D_OUT), lambda i: (0, 0)),
        ],
        out_specs=pl.BlockSpec((_BLK, D_OUT), lambda i: (i, 0)),
        out_shape=jax.ShapeDtypeStruct((n, D_OUT), jnp.float32),
        compiler_params=pltpu.CompilerParams(vmem_limit_bytes=64 * 1024 * 1024),
    )(x2, base_W, bias, router_W, a_flat, bm_flat)
    return out.reshape(b, s, D_OUT)


# speedup vs baseline: 1.0126x; 1.0015x over previous
import functools

import jax
import jax.numpy as jnp
from jax.experimental import pallas as pl
from jax.experimental.pallas import tpu as pltpu

E = 8
TOP_K = 2
RANK = 8
ALPHA = 16.0
D_IN = 2048
D_OUT = 2048
ER = E * RANK

_BLK = 512


def _body(x_ref, w_ref, b_ref, rw_ref, a_ref, bm_ref, o_ref):
    xb16 = x_ref[...].astype(jnp.bfloat16)
    dn_t = (((1,), (1,)), ((), ()))
    dn_l = (((0,), (0,)), ((), ()))

    lt = jax.lax.dot_general(rw_ref[...].astype(jnp.bfloat16), xb16, dn_t,
                             preferred_element_type=jnp.float32)
    riota = jax.lax.broadcasted_iota(jnp.int32, (E, _BLK), 0)
    v1 = jnp.max(lt, axis=0, keepdims=True)
    i1 = jnp.min(jnp.where(lt == v1, riota, E), axis=0, keepdims=True)
    masked = jnp.where(riota == i1, -jnp.inf, lt)
    v2 = jnp.max(masked, axis=0, keepdims=True)
    i2 = jnp.min(jnp.where(masked == v2, riota, E), axis=0, keepdims=True)
    ee = jnp.exp(v2 - v1)
    denom = 1.0 + ee
    p1 = (ALPHA / RANK) / denom
    p2 = (ALPHA / RANK) * ee / denom

    h = jax.lax.dot_general(a_ref[...].astype(jnp.bfloat16), xb16, dn_t,
                            preferred_element_type=jnp.float32)
    row = jax.lax.broadcasted_iota(jnp.int32, (ER, _BLK), 0) // RANK
    g = jnp.where(row == i1, p1, 0.0) + jnp.where(row == i2, p2, 0.0)
    hw = (h * g).astype(jnp.bfloat16)

    fused = jax.lax.dot_general(hw, bm_ref[...].astype(jnp.bfloat16), dn_l,
                                preferred_element_type=jnp.float32)

    w16 = w_ref[...].astype(jnp.bfloat16)
    bias = b_ref[...]
    half = _BLK // 2
    y0 = jax.lax.dot_general(xb16[:half], w16, dn_t,
                             preferred_element_type=jnp.float32)
    o_ref[:half] = y0 + (fused[:half] + bias)
    y1 = jax.lax.dot_general(xb16[half:], w16, dn_t,
                             preferred_element_type=jnp.float32)
    o_ref[half:] = y1 + (fused[half:] + bias)


@jax.jit
def kernel(x, base_W, base_b, router_W, A, Bm):
    b, s, _ = x.shape
    n = b * s
    x2 = x.reshape(n, D_IN)
    a_flat = A.reshape(ER, D_IN)
    bm_flat = jnp.transpose(Bm, (0, 2, 1)).reshape(ER, D_OUT)
    bias = base_b.reshape(1, D_OUT)

    grid = (n // _BLK,)
    out = pl.pallas_call(
        _body,
        grid=grid,
        in_specs=[
            pl.BlockSpec((_BLK, D_IN), lambda i: (i, 0)),
            pl.BlockSpec((D_OUT, D_IN), lambda i: (0, 0)),
            pl.BlockSpec((1, D_OUT), lambda i: (0, 0)),
            pl.BlockSpec((E, D_IN), lambda i: (0, 0)),
            pl.BlockSpec((ER, D_IN), lambda i: (0, 0)),
            pl.BlockSpec((ER, D_OUT), lambda i: (0, 0)),
        ],
        out_specs=pl.BlockSpec((_BLK, D_OUT), lambda i: (i, 0)),
        out_shape=jax.ShapeDtypeStruct((n, D_OUT), jnp.float32),
        compiler_params=pltpu.CompilerParams(vmem_limit_bytes=64 * 1024 * 1024),
    )(x2, base_W, bias, router_W, a_flat, bm_flat)
    return out.reshape(b, s, D_OUT)


# speedup vs baseline: 1.0140x; 1.0013x over previous
import jax
import jax.numpy as jnp
from jax.experimental import pallas as pl
from jax.experimental.pallas import tpu as pltpu

E = 8
TOP_K = 2
RANK = 8
ALPHA = 16.0
D_IN = 2048
D_OUT = 2048
ER = E * RANK

_BLK = 512


def _body(x_ref, w_ref, b_ref, rw_ref, a_ref, bm_ref, o_ref):
    xb16 = x_ref[...].astype(jnp.bfloat16)
    dn_t = (((1,), (1,)), ((), ()))
    dn_l = (((0,), (0,)), ((), ()))

    lt = jax.lax.dot_general(rw_ref[...].astype(jnp.bfloat16), xb16, dn_t,
                             preferred_element_type=jnp.float32)
    riota = jax.lax.broadcasted_iota(jnp.int32, (E, _BLK), 0)
    v1 = jnp.max(lt, axis=0, keepdims=True)
    i1 = jnp.min(jnp.where(lt == v1, riota, E), axis=0, keepdims=True)
    masked = jnp.where(riota == i1, -jnp.inf, lt)
    v2 = jnp.max(masked, axis=0, keepdims=True)
    i2 = jnp.min(jnp.where(masked == v2, riota, E), axis=0, keepdims=True)
    ee = jnp.exp(v2 - v1)
    denom = 1.0 + ee
    p1 = (ALPHA / RANK) / denom
    p2 = (ALPHA / RANK) * ee / denom

    h = jax.lax.dot_general(a_ref[...].astype(jnp.bfloat16), xb16, dn_t,
                            preferred_element_type=jnp.float32)
    row = jax.lax.broadcasted_iota(jnp.int32, (ER, _BLK), 0) // RANK
    g = jnp.where(row == i1, p1, 0.0) + jnp.where(row == i2, p2, 0.0)
    hw = (h * g).astype(jnp.bfloat16)

    fused = jax.lax.dot_general(hw, bm_ref[...].astype(jnp.bfloat16), dn_l,
                                preferred_element_type=jnp.float32)

    w16 = w_ref[...].astype(jnp.bfloat16)
    bias = b_ref[...]
    half = _BLK // 2
    y0 = jax.lax.dot_general(xb16[:half], w16, dn_t,
                             preferred_element_type=jnp.float32)
    o_ref[:half] = y0 + (fused[:half] + bias)
    y1 = jax.lax.dot_general(xb16[half:], w16, dn_t,
                             preferred_element_type=jnp.float32)
    o_ref[half:] = y1 + (fused[half:] + bias)


@jax.jit
def kernel(x, base_W, base_b, router_W, A, Bm):
    b, s, _ = x.shape
    n = b * s
    x2 = x.reshape(n, D_IN)
    a_flat = A.reshape(ER, D_IN)
    bm_flat = jnp.transpose(Bm, (0, 2, 1)).reshape(ER, D_OUT)
    bias = base_b.reshape(1, D_OUT)

    grid = (n // _BLK,)
    out = pl.pallas_call(
        _body,
        grid=grid,
        in_specs=[
            pl.BlockSpec((_BLK, D_IN), lambda i: (i, 0)),
            pl.BlockSpec((D_OUT, D_IN), lambda i: (0, 0)),
            pl.BlockSpec((1, D_OUT), lambda i: (0, 0)),
            pl.BlockSpec((E, D_IN), lambda i: (0, 0)),
            pl.BlockSpec((ER, D_IN), lambda i: (0, 0)),
            pl.BlockSpec((ER, D_OUT), lambda i: (0, 0)),
        ],
        out_specs=pl.BlockSpec((_BLK, D_OUT), lambda i: (i, 0)),
        out_shape=jax.ShapeDtypeStruct((n, D_OUT), jnp.float32),
        compiler_params=pltpu.CompilerParams(vmem_limit_bytes=64 * 1024 * 1024),
    )(x2, base_W, bias, router_W, a_flat, bm_flat)
    return out.reshape(b, s, D_OUT)
